# baseline (speedup 1.0000x reference)
import functools

import jax
import jax.numpy as jnp
from jax import lax
from jax.experimental import pallas as pl
from jax.experimental.pallas import tpu as pltpu
from jax.experimental.pallas import tpu_sc as plsc

B = 2
N = 10000
M = 10000
K = 4
D = 512
L = 16

NUM_CORES = 2
NUM_SUBCORES = 16
NW = NUM_CORES * NUM_SUBCORES

C = 16
CPB = M // C
CHUNKS = B * CPB
ITERS = -(-CHUNKS // NW)
IDX_W = K * C


def _unpool_body(xf, idxf, out,
                 idxd0, idxd1, idxa0, idxa1, g0, g1, xb0, xb1, o0, o1,
                 sem_g0, sem_g1, sem_xi0, sem_xi1,
                 sem_xo0, sem_xo1, sem_o0, sem_o1, sem_i0, sem_i1):
  g = (g0, g1)
  idxd = (idxd0, idxd1)
  idxa = (idxa0, idxa1)
  xb = (xb0, xb1)
  o = (o0, o1)
  sem_g = (sem_g0, sem_g1)
  sem_xi = (sem_xi0, sem_xi1)
  sem_xo = (sem_xo0, sem_xo1)
  sem_o = (sem_o0, sem_o1)
  sem_i = (sem_i0, sem_i1)

  w = lax.axis_index("s") * NUM_CORES + lax.axis_index("c")

  def chunk_coords(c):
    b = c // CPB
    f0 = (c - b * CPB) * C
    return b, f0

  def gather_desc(p):
    return pltpu.make_async_copy(xf.at[idxa[p]], g[p], sem_g[p])

  def idx_desc(c, p):
    cm = c - (c // CPB) * CPB
    return pltpu.make_async_copy(idxf.at[pl.ds(cm * IDX_W, IDX_W)],
                                 idxd[p], sem_i[p])

  def issue_gather(c, p):
    idx_desc(c, p).wait()
    boff = (c // CPB) * N
    for q in range(IDX_W // L):
      idxa[p][pl.ds(q * L, L)] = idxd[p][pl.ds(q * L, L)] + boff
    gather_desc(p).start()

  def xin_desc(c, p):
    b, f0 = chunk_coords(c)
    return pltpu.make_async_copy(xf.at[pl.ds(b * N + f0, C)], xb[p],
                                 sem_xi[p])

  idx_desc(w, 0).start()
  issue_gather(w, 0)
  xin_desc(w, 0).start()

  @pl.when(w + NW < CHUNKS)
  def _():
    idx_desc(w + NW, 1).start()

  def step(j, p):
    nxt = 1 - p
    c = w + j * NW
    b, f0 = chunk_coords(c)

    @pl.when(c < CHUNKS)
    def _():
      @pl.when(c + NW < CHUNKS)
      def _():
        @pl.when(j >= 1)
        def _():
          pltpu.make_async_copy(xb[nxt], out.at[pl.ds(0, C)],
                                sem_xo[nxt]).wait()
        issue_gather(c + NW, nxt)
        xin_desc(c + NW, nxt).start()
        @pl.when(c + 2 * NW < CHUNKS)
        def _():
          idx_desc(c + 2 * NW, p).start()

      xin_desc(c, p).wait()
      pltpu.make_async_copy(xb[p], out.at[pl.ds(b * (N + M) + f0, C)],
                            sem_xo[p]).start()

      gather_desc(p).wait()

      @pl.when(j >= 2)
      def _():
        pltpu.make_async_copy(o[p], out.at[pl.ds(0, C)], sem_o[p]).wait()

      def reduce_row(i4, carry):
        for r in range(4):
          i = 4 * i4 + r
          for u in range(D // L):
            ds = pl.ds(u * L, L)
            s01 = g[p][K * i, ds] + g[p][K * i + 1, ds]
            s23 = g[p][K * i + 2, ds] + g[p][K * i + 3, ds]
            o[p][i, ds] = (s01 + s23) * (1.0 / K)
        return carry

      lax.fori_loop(0, C // 4, reduce_row, 0)
      pltpu.make_async_copy(o[p], out.at[pl.ds(b * (N + M) + N + f0, C)],
                            sem_o[p]).start()

  def fori_body(j2, carry):
    for p in range(2):
      step(2 * j2 + p, p)
    return carry

  lax.fori_loop(0, ITERS // 2, fori_body, 0)

  for p in range(2):
    pltpu.make_async_copy(xb[p], out.at[pl.ds(0, C)], sem_xo[p]).wait()
    pltpu.make_async_copy(o[p], out.at[pl.ds(0, C)], sem_o[p]).wait()


@jax.jit
def kernel(x, pool_idx):
  xf = x.reshape(B * N, D)
  idxf = pool_idx.astype(jnp.int32).reshape(M * K)

  mesh = plsc.VectorSubcoreMesh(
      core_axis_name="c", subcore_axis_name="s",
      num_cores=NUM_CORES, num_subcores=NUM_SUBCORES)

  out = pl.kernel(
      _unpool_body,
      out_type=jax.ShapeDtypeStruct((B * (N + M), D), jnp.float32),
      mesh=mesh,
      scratch_types=[
          pltpu.VMEM((IDX_W,), jnp.int32),
          pltpu.VMEM((IDX_W,), jnp.int32),
          pltpu.VMEM((IDX_W,), jnp.int32),
          pltpu.VMEM((IDX_W,), jnp.int32),
          pltpu.VMEM((IDX_W, D), jnp.float32),
          pltpu.VMEM((IDX_W, D), jnp.float32),
          pltpu.VMEM((C, D), jnp.float32),
          pltpu.VMEM((C, D), jnp.float32),
          pltpu.VMEM((C, D), jnp.float32),
          pltpu.VMEM((C, D), jnp.float32),
          pltpu.SemaphoreType.DMA,
          pltpu.SemaphoreType.DMA,
          pltpu.SemaphoreType.DMA,
          pltpu.SemaphoreType.DMA,
          pltpu.SemaphoreType.DMA,
          pltpu.SemaphoreType.DMA,
          pltpu.SemaphoreType.DMA,
          pltpu.SemaphoreType.DMA,
          pltpu.SemaphoreType.DMA,
          pltpu.SemaphoreType.DMA,
      ],
  )(xf, idxf)

  return out.reshape(B, N + M, D)


# speedup vs baseline: 1.0482x; 1.0482x over previous
import functools

import jax
import jax.numpy as jnp
from jax import lax
from jax.experimental import pallas as pl
from jax.experimental.pallas import tpu as pltpu
from jax.experimental.pallas import tpu_sc as plsc

B = 2
N = 10000
M = 10000
K = 4
D = 512
L = 16

NUM_CORES = 2
NUM_SUBCORES = 16
NW = NUM_CORES * NUM_SUBCORES

C = 16
CPB = M // C
CHUNKS = B * CPB
ITERS = -(-CHUNKS // NW)
IDX_W = K * C


def _unpool_body(xf, idxf, out,
                 idxd0, idxd1, idxa0, idxa1, g0, g1, shx, o0, o1,
                 sem_g0, sem_g1, sem_xi0, sem_xi1,
                 sem_xo0, sem_xo1, sem_o0, sem_o1, sem_i0, sem_i1):
  g = (g0, g1)
  idxd = (idxd0, idxd1)
  idxa = (idxa0, idxa1)
  o = (o0, o1)
  sem_g = (sem_g0, sem_g1)
  sem_xi = (sem_xi0, sem_xi1)
  sem_xo = (sem_xo0, sem_xo1)
  sem_o = (sem_o0, sem_o1)
  sem_i = (sem_i0, sem_i1)

  sid = lax.axis_index("s")
  w = sid * NUM_CORES + lax.axis_index("c")

  def chunk_coords(c):
    b = c // CPB
    f0 = (c - b * CPB) * C
    return b, f0

  def gather_desc(p):
    return pltpu.make_async_copy(xf.at[idxa[p]], g[p], sem_g[p])

  def idx_desc(c, p):
    cm = c - (c // CPB) * CPB
    return pltpu.make_async_copy(idxf.at[pl.ds(cm * IDX_W, IDX_W)],
                                 idxd[p], sem_i[p])

  def issue_gather(c, p):
    idx_desc(c, p).wait()
    boff = (c // CPB) * N
    for q in range(IDX_W // L):
      idxa[p][pl.ds(q * L, L)] = idxd[p][pl.ds(q * L, L)] + boff
    gather_desc(p).start()

  def xin_desc(c, p):
    b, f0 = chunk_coords(c)
    return pltpu.make_async_copy(xf.at[pl.ds(b * N + f0, C)],
                                 shx.at[sid, p], sem_xi[p])

  idx_desc(w, 0).start()
  issue_gather(w, 0)
  xin_desc(w, 0).start()

  @pl.when(w + NW < CHUNKS)
  def _():
    idx_desc(w + NW, 1).start()

  def step(j, p):
    nxt = 1 - p
    c = w + j * NW
    b, f0 = chunk_coords(c)

    @pl.when(c < CHUNKS)
    def _():
      @pl.when(c + NW < CHUNKS)
      def _():
        @pl.when(j >= 1)
        def _():
          pltpu.make_async_copy(shx.at[sid, nxt], out.at[pl.ds(0, C)],
                                sem_xo[nxt]).wait()
        issue_gather(c + NW, nxt)
        xin_desc(c + NW, nxt).start()
        @pl.when(c + 2 * NW < CHUNKS)
        def _():
          idx_desc(c + 2 * NW, p).start()

      xin_desc(c, p).wait()
      pltpu.make_async_copy(shx.at[sid, p],
                            out.at[pl.ds(b * (N + M) + f0, C)],
                            sem_xo[p]).start()

      gather_desc(p).wait()

      @pl.when(j >= 2)
      def _():
        pltpu.make_async_copy(o[p], out.at[pl.ds(0, C)], sem_o[p]).wait()

      def reduce_row(i2, carry):
        for r in range(2):
          i = 2 * i2 + r
          for u in range(D // L):
            ds = pl.ds(u * L, L)
            acc = g[p][K * i, ds]
            for kk in range(1, K):
              acc = acc + g[p][K * i + kk, ds]
            o[p][i, ds] = acc * (1.0 / K)
        return carry

      lax.fori_loop(0, C // 2, reduce_row, 0)
      pltpu.make_async_copy(o[p], out.at[pl.ds(b * (N + M) + N + f0, C)],
                            sem_o[p]).start()

  def fori_body(j2, carry):
    for p in range(2):
      step(2 * j2 + p, p)
    return carry

  lax.fori_loop(0, ITERS // 2, fori_body, 0)

  for p in range(2):
    pltpu.make_async_copy(shx.at[sid, p], out.at[pl.ds(0, C)],
                          sem_xo[p]).wait()
    pltpu.make_async_copy(o[p], out.at[pl.ds(0, C)], sem_o[p]).wait()


@jax.jit
def kernel(x, pool_idx):
  xf = x.reshape(B * N, D)
  idxf = pool_idx.astype(jnp.int32).reshape(M * K)

  mesh = plsc.VectorSubcoreMesh(
      core_axis_name="c", subcore_axis_name="s",
      num_cores=NUM_CORES, num_subcores=NUM_SUBCORES)

  out = pl.kernel(
      _unpool_body,
      out_type=jax.ShapeDtypeStruct((B * (N + M), D), jnp.float32),
      mesh=mesh,
      scratch_types=[
          pltpu.VMEM((IDX_W,), jnp.int32),
          pltpu.VMEM((IDX_W,), jnp.int32),
          pltpu.VMEM((IDX_W,), jnp.int32),
          pltpu.VMEM((IDX_W,), jnp.int32),
          pltpu.VMEM((IDX_W, D), jnp.float32),
          pltpu.VMEM((IDX_W, D), jnp.float32),
          pltpu.VMEM_SHARED((NUM_SUBCORES, 2, C, D),
                            jnp.float32),
          pltpu.VMEM((C, D), jnp.float32),
          pltpu.VMEM((C, D), jnp.float32),
          pltpu.SemaphoreType.DMA,
          pltpu.SemaphoreType.DMA,
          pltpu.SemaphoreType.DMA,
          pltpu.SemaphoreType.DMA,
          pltpu.SemaphoreType.DMA,
          pltpu.SemaphoreType.DMA,
          pltpu.SemaphoreType.DMA,
          pltpu.SemaphoreType.DMA,
          pltpu.SemaphoreType.DMA,
          pltpu.SemaphoreType.DMA,
      ],
  )(xf, idxf)

  return out.reshape(B, N + M, D)


# speedup vs baseline: 1.6864x; 1.6089x over previous
import functools

import jax
import jax.numpy as jnp
from jax import lax
from jax.experimental import pallas as pl
from jax.experimental.pallas import tpu as pltpu
from jax.experimental.pallas import tpu_sc as plsc

B = 2
N = 10000
M = 10000
K = 4
D = 512
L = 16

NUM_CORES = 2
NUM_SUBCORES = 16
NW = NUM_CORES * NUM_SUBCORES

C = 16
CPB = M // C
CHUNKS = B * CPB
ITERS = -(-CHUNKS // NW)
IDX_W = K * C


def _unpool_body(xf, idxf, out,
                 idxd0, idxd1, idxa0, idxa1, g0, g1, shx, o0, o1,
                 sem_g0, sem_g1, sem_xi0, sem_xi1,
                 sem_xo0, sem_xo1, sem_o0, sem_o1, sem_i0, sem_i1):
  g = (g0, g1)
  idxd = (idxd0, idxd1)
  idxa = (idxa0, idxa1)
  o = (o0, o1)
  sem_g = (sem_g0, sem_g1)
  sem_xi = (sem_xi0, sem_xi1)
  sem_xo = (sem_xo0, sem_xo1)
  sem_o = (sem_o0, sem_o1)
  sem_i = (sem_i0, sem_i1)

  sid = lax.axis_index("s")
  w = sid * NUM_CORES + lax.axis_index("c")

  def chunk_coords(c):
    b = c // CPB
    f0 = (c - b * CPB) * C
    return b, f0

  def gather_desc(p):
    return pltpu.make_async_copy(xf.at[idxa[p]], g[p], sem_g[p])

  def idx_desc(c, p):
    cm = c - (c // CPB) * CPB
    return pltpu.make_async_copy(idxf.at[pl.ds(cm * IDX_W, IDX_W)],
                                 idxd[p], sem_i[p])

  def issue_gather(c, p):
    idx_desc(c, p).wait()
    boff = (c // CPB) * N
    for q in range(IDX_W // L):
      idxa[p][pl.ds(q * L, L)] = idxd[p][pl.ds(q * L, L)] + boff
    gather_desc(p).start()

  def xin_desc(c, p):
    b, f0 = chunk_coords(c)
    return pltpu.make_async_copy(xf.at[pl.ds(b * N + f0, C)],
                                 shx.at[sid, p], sem_xi[p])

  idx_desc(w, 0).start()
  issue_gather(w, 0)
  xin_desc(w, 0).start()

  @pl.when(w + NW < CHUNKS)
  def _():
    idx_desc(w + NW, 1).start()

  def step(j, p):
    nxt = 1 - p
    c = w + j * NW
    b, f0 = chunk_coords(c)

    @pl.when(c < CHUNKS)
    def _():
      @pl.when(c + NW < CHUNKS)
      def _():
        @pl.when(j >= 1)
        def _():
          pltpu.make_async_copy(shx.at[sid, nxt], out.at[pl.ds(0, C)],
                                sem_xo[nxt]).wait()
        issue_gather(c + NW, nxt)
        xin_desc(c + NW, nxt).start()
        @pl.when(c + 2 * NW < CHUNKS)
        def _():
          idx_desc(c + 2 * NW, p).start()

      xin_desc(c, p).wait()
      pltpu.make_async_copy(shx.at[sid, p],
                            out.at[pl.ds(b * (N + M) + f0, C)],
                            sem_xo[p]).start()

      gather_desc(p).wait()

      @pl.when(j >= 2)
      def _():
        pltpu.make_async_copy(o[p], out.at[pl.ds(0, C)], sem_o[p]).wait()

      @plsc.parallel_loop(0, C, 1, unroll=2)
      def _(i):
        for u in range(D // L):
          ds = pl.ds(u * L, L)
          acc = g[p][K * i, ds]
          for kk in range(1, K):
            acc = acc + g[p][K * i + kk, ds]
          o[p][i, ds] = acc * (1.0 / K)
      pltpu.make_async_copy(o[p], out.at[pl.ds(b * (N + M) + N + f0, C)],
                            sem_o[p]).start()

  def fori_body(j2, carry):
    for p in range(2):
      step(2 * j2 + p, p)
    return carry

  lax.fori_loop(0, ITERS // 2, fori_body, 0)

  for p in range(2):
    pltpu.make_async_copy(shx.at[sid, p], out.at[pl.ds(0, C)],
                          sem_xo[p]).wait()
    pltpu.make_async_copy(o[p], out.at[pl.ds(0, C)], sem_o[p]).wait()


@jax.jit
def kernel(x, pool_idx):
  xf = x.reshape(B * N, D)
  idxf = pool_idx.astype(jnp.int32).reshape(M * K)

  mesh = plsc.VectorSubcoreMesh(
      core_axis_name="c", subcore_axis_name="s",
      num_cores=NUM_CORES, num_subcores=NUM_SUBCORES)

  out = pl.kernel(
      _unpool_body,
      out_type=jax.ShapeDtypeStruct((B * (N + M), D), jnp.float32),
      mesh=mesh,
      scratch_types=[
          pltpu.VMEM((IDX_W,), jnp.int32),
          pltpu.VMEM((IDX_W,), jnp.int32),
          pltpu.VMEM((IDX_W,), jnp.int32),
          pltpu.VMEM((IDX_W,), jnp.int32),
          pltpu.VMEM((IDX_W, D), jnp.float32),
          pltpu.VMEM((IDX_W, D), jnp.float32),
          pltpu.VMEM_SHARED((NUM_SUBCORES, 2, C, D),
                            jnp.float32),
          pltpu.VMEM((C, D), jnp.float32),
          pltpu.VMEM((C, D), jnp.float32),
          pltpu.SemaphoreType.DMA,
          pltpu.SemaphoreType.DMA,
          pltpu.SemaphoreType.DMA,
          pltpu.SemaphoreType.DMA,
          pltpu.SemaphoreType.DMA,
          pltpu.SemaphoreType.DMA,
          pltpu.SemaphoreType.DMA,
          pltpu.SemaphoreType.DMA,
          pltpu.SemaphoreType.DMA,
          pltpu.SemaphoreType.DMA,
      ],
  )(xf, idxf)

  return out.reshape(B, N + M, D)
